# baseline (speedup 1.0000x reference)
import jax
import jax.numpy as jnp
from jax.experimental import pallas as pl

_BM = 200


def _support_body(x_ref, w_ref, o_ref):
    o_ref[:] = jnp.dot(x_ref[:], w_ref[:], preferred_element_type=jnp.float32)


def _fused_body(adj_ref, adjw_ref, sup_ref, mlpwt_ref, b_ref, o_ref):
    a = adj_ref[:] + adjw_ref[:]
    h = jnp.dot(a, sup_ref[:], preferred_element_type=jnp.float32)
    norm = jnp.maximum(jnp.sqrt(jnp.sum(h * h, axis=-1, keepdims=True)), 1e-12)
    h = jnp.maximum(h / norm, 0.0)
    o_ref[:] = jnp.dot(h, mlpwt_ref[:], preferred_element_type=jnp.float32) + b_ref[:]


def kernel(x, adj, adj_w, W, mlp_W, mlp_b):
    n, d = x.shape
    nclass = mlp_W.shape[0]

    support = pl.pallas_call(
        _support_body,
        out_shape=jax.ShapeDtypeStruct((n, d), jnp.float32),
    )(x, W)

    mlp_wt = mlp_W.T
    b2 = mlp_b.reshape(1, nclass)

    return pl.pallas_call(
        _fused_body,
        grid=(n // _BM,),
        in_specs=[
            pl.BlockSpec((_BM, n), lambda i: (i, 0)),
            pl.BlockSpec((_BM, n), lambda i: (i, 0)),
            pl.BlockSpec((n, d), lambda i: (0, 0)),
            pl.BlockSpec((d, nclass), lambda i: (0, 0)),
            pl.BlockSpec((1, nclass), lambda i: (0, 0)),
        ],
        out_specs=pl.BlockSpec((_BM, nclass), lambda i: (i, 0)),
        out_shape=jax.ShapeDtypeStruct((n, nclass), jnp.float32),
    )(adj, adj_w, support, mlp_wt, b2)


# speedup vs baseline: 1.0223x; 1.0223x over previous
import jax
import jax.numpy as jnp
from jax.experimental import pallas as pl

_BM = 200


def _fused_body(adj_ref, adjw_ref, x_ref, w_ref, mlpwt_ref, b_ref, o_ref):
    a = adj_ref[:] + adjw_ref[:]
    h = jnp.dot(a, x_ref[:], preferred_element_type=jnp.float32)
    h = jnp.dot(h, w_ref[:], preferred_element_type=jnp.float32)
    norm = jnp.maximum(jnp.sqrt(jnp.sum(h * h, axis=-1, keepdims=True)), 1e-12)
    h = jnp.maximum(h / norm, 0.0)
    o_ref[:] = jnp.dot(h, mlpwt_ref[:], preferred_element_type=jnp.float32) + b_ref[:]


def kernel(x, adj, adj_w, W, mlp_W, mlp_b):
    n, d = x.shape
    nclass = mlp_W.shape[0]
    mlp_wt = mlp_W.T
    b2 = mlp_b.reshape(1, nclass)

    return pl.pallas_call(
        _fused_body,
        grid=(n // _BM,),
        in_specs=[
            pl.BlockSpec((_BM, n), lambda i: (i, 0)),
            pl.BlockSpec((_BM, n), lambda i: (i, 0)),
            pl.BlockSpec((n, d), lambda i: (0, 0)),
            pl.BlockSpec((d, d), lambda i: (0, 0)),
            pl.BlockSpec((d, nclass), lambda i: (0, 0)),
            pl.BlockSpec((1, nclass), lambda i: (0, 0)),
        ],
        out_specs=pl.BlockSpec((_BM, nclass), lambda i: (i, 0)),
        out_shape=jax.ShapeDtypeStruct((n, nclass), jnp.float32),
    )(adj, adj_w, x, W, mlp_wt, b2)


# speedup vs baseline: 1.0262x; 1.0039x over previous
import jax
import jax.numpy as jnp
from jax.experimental import pallas as pl

_BM = 200


def _fused_body(adj_ref, adjw_ref, x_ref, w_ref, mlpw_ref, b_ref, o_ref):
    a = adj_ref[:] + adjw_ref[:]
    h = jnp.dot(a, x_ref[:], preferred_element_type=jnp.float32)
    h = jnp.dot(h, w_ref[:], preferred_element_type=jnp.float32)
    norm = jnp.maximum(jnp.sqrt(jnp.sum(h * h, axis=-1, keepdims=True)), 1e-12)
    h = jnp.maximum(h / norm, 0.0)
    o_ref[:] = jax.lax.dot_general(
        h, mlpw_ref[:], (((1,), (1,)), ((), ())),
        preferred_element_type=jnp.float32) + b_ref[:]


def kernel(x, adj, adj_w, W, mlp_W, mlp_b):
    n, d = x.shape
    nclass = mlp_W.shape[0]
    b2 = mlp_b.reshape(1, nclass)

    return pl.pallas_call(
        _fused_body,
        grid=(n // _BM,),
        in_specs=[
            pl.BlockSpec((_BM, n), lambda i: (i, 0)),
            pl.BlockSpec((_BM, n), lambda i: (i, 0)),
            pl.BlockSpec((n, d), lambda i: (0, 0)),
            pl.BlockSpec((d, d), lambda i: (0, 0)),
            pl.BlockSpec((nclass, d), lambda i: (0, 0)),
            pl.BlockSpec((1, nclass), lambda i: (0, 0)),
        ],
        out_specs=pl.BlockSpec((_BM, nclass), lambda i: (i, 0)),
        out_shape=jax.ShapeDtypeStruct((n, nclass), jnp.float32),
    )(adj, adj_w, x, W, mlp_W, b2)
